# baseline (speedup 1.0000x reference)
import jax
import jax.numpy as jnp
from jax.experimental import pallas as pl
from jax.experimental.pallas import tpu as pltpu

_B, _C, _H, _W = 2, 5, 721, 1440
_K = 50
_DX = 25000.0
_DY = 25000.0
_VORT_THR = 1.4e-4
_FILL = -9999.0
_NEG = -3.0e38
_BIGF = 3.0e38
_HP = 728


def _rshift(a, s):
    return jnp.concatenate([a[s:], a[:s]], axis=0) if s > 0 else \
        jnp.concatenate([a[_H + s:], a[:_H + s]], axis=0)


def _tc_body(x_ref, out_ref, xs2_ref, xs3_ref, mslp_ref, w10p_ref, m_ref,
             sem1, sem2):
    b = pl.program_id(0)
    h1 = pltpu.make_async_copy(x_ref.at[b, pl.ds(3, 2)], xs2_ref, sem1)
    h2 = pltpu.make_async_copy(x_ref.at[b, pl.ds(0, 3)], xs3_ref, sem2)
    h1.start()
    h2.start()
    h1.wait()
    u850 = xs2_ref[0]
    v850 = xs2_ref[1]

    du = jnp.concatenate(
        [u850[1:2] - u850[0:1],
         (u850[2:] - u850[:-2]) / 2.0,
         u850[_H - 1:_H] - u850[_H - 2:_H - 1]], axis=0) / _DX
    dv = jnp.concatenate(
        [v850[:, 1:2] - v850[:, 0:1],
         (v850[:, 2:] - v850[:, :-2]) / 2.0,
         v850[:, _W - 1:_W] - v850[:, _W - 2:_W - 1]], axis=1) / _DY
    vort = du + dv

    m1 = jnp.maximum(jnp.maximum(vort, _rshift(vort, 1)), _rshift(vort, -1))
    lf = jnp.concatenate([m1[:, 1:], m1[:, :1]], axis=1)
    rt = jnp.concatenate([m1[:, _W - 1:], m1[:, :_W - 1]], axis=1)
    m2 = jnp.maximum(jnp.maximum(m1, lf), rt)
    is_peak = (vort >= m2) & (vort > _VORT_THR)
    masked = jnp.concatenate(
        [jnp.where(is_peak, vort, _NEG),
         jnp.full((_HP - _H, _W), _NEG, jnp.float32)], axis=0)
    m_ref[:, :] = masked
    rmax = jnp.max(masked.reshape(_HP // 8, 8, _W), axis=2)

    h2.wait()
    u10 = xs3_ref[0]
    v10 = xs3_ref[1]
    msl = xs3_ref[2]
    w10 = jnp.sqrt(u10 * u10 + v10 * v10)
    padB = jnp.full((_HP - _H, _W), _BIGF, jnp.float32)
    mp = jnp.minimum(msl, jnp.minimum(_rshift(msl, 1), _rshift(msl, -1)))
    mp = jnp.minimum(mp, jnp.minimum(_rshift(msl, 2), _rshift(msl, -2)))
    mslp_ref[:, :] = jnp.concatenate([mp, padB], axis=0).astype(jnp.bfloat16)
    wp = jnp.maximum(w10, jnp.maximum(_rshift(w10, 1), _rshift(w10, -1)))
    wp = jnp.maximum(wp, jnp.maximum(_rshift(w10, 2), _rshift(w10, -2)))
    w10p_ref[:, :] = jnp.concatenate([wp, -padB], axis=0).astype(jnp.bfloat16)

    iota_g = jax.lax.broadcasted_iota(jnp.int32, (_HP // 8, 8), 0)
    iota_l = jax.lax.broadcasted_iota(jnp.int32, (_HP // 8, 8), 1)
    iota_gl = iota_g * 8 + iota_l
    iota_r8 = jax.lax.broadcasted_iota(jnp.int32, (8, 1), 0)
    iota_c8 = jax.lax.broadcasted_iota(jnp.int32, (8, _W), 1)
    c4 = jax.lax.broadcasted_iota(jnp.int32, (1, 4), 1)

    for k in range(_K):
        rm = jnp.max(rmax)
        ri = jnp.min(jnp.where(rmax == rm, iota_gl, _HP))
        base = pl.multiple_of((ri // 8) * 8, 8)
        off = ri - base
        blk = m_ref[pl.ds(base, 8), :]
        rowsel = iota_r8 == off
        vals = jnp.where(rowsel, blk, _NEG)
        ci = jnp.min(jnp.where(vals == rm, iota_c8, _W))
        eqci = iota_c8 == ci
        m_ref[pl.ds(base, 8), :] = jnp.where(rowsel & eqci, _NEG, blk)
        newvals = jnp.where(eqci, _NEG, vals)
        newrowmax = jnp.max(newvals)
        rmax = jnp.where(iota_gl == ri, newrowmax, rmax)
        d = iota_c8 - ci + 2
        colmask = ((d >= 0) & (d < 5)) | (d >= _W) | (d < 5 - _W)
        wmask = rowsel & colmask
        msl8 = mslp_ref[pl.ds(base, 8), :].astype(jnp.float32)
        w108 = w10p_ref[pl.ds(base, 8), :].astype(jnp.float32)
        mslc = jnp.min(jnp.where(wmask, msl8, _BIGF))
        w10c = jnp.max(jnp.where(wmask, w108, -_BIGF))
        valid = rm > _VORT_THR
        latv = jnp.where(valid, 90.0 - 0.25 * ri.astype(jnp.float32), _FILL)
        lonv = jnp.where(valid, 0.25 * ci.astype(jnp.float32), _FILL)
        mslv = jnp.where(valid, mslc, _FILL)
        w10v = jnp.where(valid, w10c, _FILL)
        vec = jnp.where(c4 == 0, latv,
                        jnp.where(c4 == 1, lonv,
                                  jnp.where(c4 == 2, mslv, w10v)))
        out_ref[0, k:k + 1, :] = vec


def kernel(x):
    return pl.pallas_call(
        _tc_body,
        grid=(_B,),
        in_specs=[pl.BlockSpec(memory_space=pl.ANY)],
        out_specs=pl.BlockSpec((1, _K, 4), lambda i: (i, 0, 0)),
        out_shape=jax.ShapeDtypeStruct((_B, _K, 4), jnp.float32),
        scratch_shapes=[
            pltpu.VMEM((2, _H, _W), jnp.float32),
            pltpu.VMEM((3, _H, _W), jnp.float32),
            pltpu.VMEM((_HP, _W), jnp.bfloat16),
            pltpu.VMEM((_HP, _W), jnp.bfloat16),
            pltpu.VMEM((_HP, _W), jnp.float32),
            pltpu.SemaphoreType.DMA,
            pltpu.SemaphoreType.DMA,
        ],
    )(x)


# speedup vs baseline: 1.0607x; 1.0607x over previous
import jax
import jax.numpy as jnp
from jax.experimental import pallas as pl
from jax.experimental.pallas import tpu as pltpu

_B, _C, _H, _W = 2, 5, 721, 1440
_K = 50
_DX = 25000.0
_DY = 25000.0
_VORT_THR = 1.4e-4
_FILL = -9999.0
_NEG = -3.0e38
_BIGF = 3.0e38
_HP = 728


def _rshift(a, s):
    return jnp.concatenate([a[s:], a[:s]], axis=0) if s > 0 else \
        jnp.concatenate([a[_H + s:], a[:_H + s]], axis=0)


def _tc_body(x_ref, out_ref, xs2_ref, xs3_ref, mslp_ref, w10p_ref, m_ref,
             sem1, sem2):
    b = pl.program_id(0)
    h1 = pltpu.make_async_copy(x_ref.at[b, pl.ds(3, 2)], xs2_ref, sem1)
    h2 = pltpu.make_async_copy(x_ref.at[b, pl.ds(0, 3)], xs3_ref, sem2)
    h1.start()
    h2.start()
    h1.wait()
    u850 = xs2_ref[0]
    v850 = xs2_ref[1]

    du = jnp.concatenate(
        [u850[1:2] - u850[0:1],
         (u850[2:] - u850[:-2]) / 2.0,
         u850[_H - 1:_H] - u850[_H - 2:_H - 1]], axis=0) / _DX
    dv = jnp.concatenate(
        [v850[:, 1:2] - v850[:, 0:1],
         (v850[:, 2:] - v850[:, :-2]) / 2.0,
         v850[:, _W - 1:_W] - v850[:, _W - 2:_W - 1]], axis=1) / _DY
    vort = du + dv

    m1 = jnp.maximum(jnp.maximum(vort, _rshift(vort, 1)), _rshift(vort, -1))
    lf = jnp.concatenate([m1[:, 1:], m1[:, :1]], axis=1)
    rt = jnp.concatenate([m1[:, _W - 1:], m1[:, :_W - 1]], axis=1)
    m2 = jnp.maximum(jnp.maximum(m1, lf), rt)
    is_peak = (vort >= m2) & (vort > _VORT_THR)
    masked = jnp.concatenate(
        [jnp.where(is_peak, vort, _NEG),
         jnp.full((_HP - _H, _W), _NEG, jnp.float32)], axis=0)
    m_ref[:, :] = masked
    rmax = jnp.max(masked, axis=1, keepdims=True)

    h2.wait()
    u10 = xs3_ref[0]
    v10 = xs3_ref[1]
    msl = xs3_ref[2]
    w10 = jnp.sqrt(u10 * u10 + v10 * v10)
    padB = jnp.full((_HP - _H, _W), _BIGF, jnp.float32)
    mp = jnp.minimum(msl, jnp.minimum(_rshift(msl, 1), _rshift(msl, -1)))
    mp = jnp.minimum(mp, jnp.minimum(_rshift(msl, 2), _rshift(msl, -2)))
    mslp_ref[:, :] = jnp.concatenate([mp, padB], axis=0).astype(jnp.bfloat16)
    wp = jnp.maximum(w10, jnp.maximum(_rshift(w10, 1), _rshift(w10, -1)))
    wp = jnp.maximum(wp, jnp.maximum(_rshift(w10, 2), _rshift(w10, -2)))
    w10p_ref[:, :] = jnp.concatenate([wp, -padB], axis=0).astype(jnp.bfloat16)

    iota_r = jax.lax.broadcasted_iota(jnp.int32, (_HP, 1), 0)
    iota_r8 = jax.lax.broadcasted_iota(jnp.int32, (8, 1), 0)
    iota_c8 = jax.lax.broadcasted_iota(jnp.int32, (8, _W), 1)
    c4 = jax.lax.broadcasted_iota(jnp.int32, (1, 4), 1)

    for k in range(_K):
        rm = jnp.max(rmax)
        ri = jnp.min(jnp.where(rmax == rm, iota_r, _HP))
        base = pl.multiple_of((ri // 8) * 8, 8)
        off = ri - base
        blk = m_ref[pl.ds(base, 8), :]
        rowsel = iota_r8 == off
        vals = jnp.where(rowsel, blk, _NEG)
        ci = jnp.min(jnp.where(vals == rm, iota_c8, _W))
        eqci = iota_c8 == ci
        m_ref[pl.ds(base, 8), :] = jnp.where(rowsel & eqci, _NEG, blk)
        newvals = jnp.where(eqci, _NEG, vals)
        newrowmax = jnp.max(newvals)
        rmax = jnp.where(iota_r == ri, newrowmax, rmax)
        d = iota_c8 - ci + 2
        colmask = ((d >= 0) & (d < 5)) | (d >= _W) | (d < 5 - _W)
        wmask = rowsel & colmask
        msl8 = mslp_ref[pl.ds(base, 8), :].astype(jnp.float32)
        w108 = w10p_ref[pl.ds(base, 8), :].astype(jnp.float32)
        mslc = jnp.min(jnp.where(wmask, msl8, _BIGF))
        w10c = jnp.max(jnp.where(wmask, w108, -_BIGF))
        valid = rm > _VORT_THR
        latv = jnp.where(valid, 90.0 - 0.25 * ri.astype(jnp.float32), _FILL)
        lonv = jnp.where(valid, 0.25 * ci.astype(jnp.float32), _FILL)
        mslv = jnp.where(valid, mslc, _FILL)
        w10v = jnp.where(valid, w10c, _FILL)
        vec = jnp.where(c4 == 0, latv,
                        jnp.where(c4 == 1, lonv,
                                  jnp.where(c4 == 2, mslv, w10v)))
        out_ref[0, k:k + 1, :] = vec


def kernel(x):
    return pl.pallas_call(
        _tc_body,
        grid=(_B,),
        in_specs=[pl.BlockSpec(memory_space=pl.ANY)],
        out_specs=pl.BlockSpec((1, _K, 4), lambda i: (i, 0, 0)),
        out_shape=jax.ShapeDtypeStruct((_B, _K, 4), jnp.float32),
        scratch_shapes=[
            pltpu.VMEM((2, _H, _W), jnp.float32),
            pltpu.VMEM((3, _H, _W), jnp.float32),
            pltpu.VMEM((_HP, _W), jnp.bfloat16),
            pltpu.VMEM((_HP, _W), jnp.bfloat16),
            pltpu.VMEM((_HP, _W), jnp.float32),
            pltpu.SemaphoreType.DMA,
            pltpu.SemaphoreType.DMA,
        ],
    )(x)
